# baseline (speedup 1.0000x reference)
import functools

import jax
import jax.numpy as jnp
from jax import lax
from jax.experimental import pallas as pl
from jax.experimental.pallas import tpu as pltpu
from jax.experimental.pallas import tpu_sc as plsc

_NC = 2
_NS = 16


def _sc_gather(hours, pe, B, T, F):
    mesh = plsc.VectorSubcoreMesh(
        core_axis_name="c", subcore_axis_name="s", num_cores=_NC, num_subcores=_NS
    )

    @functools.partial(
        pl.kernel,
        out_type=jax.ShapeDtypeStruct((B, T, F), jnp.float32),
        mesh=mesh,
        scratch_types=[
            pltpu.VMEM((T,), jnp.int32),
            pltpu.VMEM((T, F), jnp.float32),
            pltpu.SemaphoreType.DMA,
        ],
        compiler_params=pltpu.CompilerParams(use_tc_tiling_on_sc=False),
    )
    def gather_kernel(hours_hbm, pe_hbm, pos_hbm, idx_v, rows_v, sem):
        wid = lax.axis_index("s") * _NC + lax.axis_index("c")

        @pl.when(wid < B)
        def _():
            pltpu.sync_copy(hours_hbm.at[wid], idx_v)
            pltpu.async_copy(pe_hbm.at[idx_v], rows_v, sem).wait()
            pltpu.sync_copy(rows_v, pos_hbm.at[wid])

    return gather_kernel(hours, pe)


def _tc_broadcast(pos, B, N, T, F, n_block):
    pos4 = pos.reshape(B, 1, T, F)

    def body(pos_ref, out_ref):
        out_ref[...] = jnp.broadcast_to(pos_ref[...], out_ref.shape)

    return pl.pallas_call(
        body,
        grid=(B, N // n_block),
        in_specs=[pl.BlockSpec((1, 1, T, F), lambda b, n: (b, 0, 0, 0))],
        out_specs=pl.BlockSpec((1, n_block, T, F), lambda b, n: (b, n, 0, 0)),
        out_shape=jax.ShapeDtypeStruct((B, N, T, F), jnp.float32),
    )(pos4)


def _tc_gather_broadcast(hours, pe, B, N, T, F):
    V = pe.shape[0]

    def body(hours_smem, pe_ref, out_ref):
        b = pl.program_id(0)
        t = pl.program_id(1)
        row = pe_ref[hours_smem[b, t], :]
        out_ref[...] = jnp.broadcast_to(row[None, None, :, None], (1, 1, F, N))

    grid_spec = pltpu.PrefetchScalarGridSpec(
        num_scalar_prefetch=1,
        grid=(B, T),
        in_specs=[pl.BlockSpec((V, F), lambda b, t, hrs: (0, 0))],
        out_specs=pl.BlockSpec((1, 1, F, N), lambda b, t, hrs: (b, t, 0, 0)),
    )
    out_t = pl.pallas_call(
        body,
        grid_spec=grid_spec,
        out_shape=jax.ShapeDtypeStruct((B, T, F, N), jnp.float32),
    )(hours, pe)
    return jnp.transpose(out_t, (0, 3, 1, 2))


def kernel(x, hours, pe):
    B, N, T, F = x.shape
    hours = hours.astype(jnp.int32)
    return _tc_gather_broadcast(hours, pe, B, N, T, F)


# speedup vs baseline: 4.8095x; 4.8095x over previous
import functools

import jax
import jax.numpy as jnp
from jax import lax
from jax.experimental import pallas as pl
from jax.experimental.pallas import tpu as pltpu
from jax.experimental.pallas import tpu_sc as plsc

_NC = 2
_NS = 16


def _sc_gather(hours, pe, B, T, F):
    mesh = plsc.VectorSubcoreMesh(
        core_axis_name="c", subcore_axis_name="s", num_cores=_NC, num_subcores=_NS
    )

    @functools.partial(
        pl.kernel,
        out_type=jax.ShapeDtypeStruct((B, T, F), jnp.float32),
        mesh=mesh,
        scratch_types=[
            pltpu.VMEM((T,), jnp.int32),
            pltpu.VMEM((T, F), jnp.float32),
            pltpu.SemaphoreType.DMA,
        ],
        compiler_params=pltpu.CompilerParams(use_tc_tiling_on_sc=False),
    )
    def gather_kernel(hours_hbm, pe_hbm, pos_hbm, idx_v, rows_v, sem):
        wid = lax.axis_index("s") * _NC + lax.axis_index("c")

        @pl.when(wid < B)
        def _():
            pltpu.sync_copy(hours_hbm.at[wid], idx_v)
            pltpu.async_copy(pe_hbm.at[idx_v], rows_v, sem).wait()
            pltpu.sync_copy(rows_v, pos_hbm.at[wid])

    return gather_kernel(hours, pe)


def _tc_broadcast(pos, B, N, T, F, n_block):
    pos4 = pos.reshape(B, 1, T, F)

    def body(pos_ref, out_ref):
        out_ref[...] = jnp.broadcast_to(pos_ref[...], out_ref.shape)

    return pl.pallas_call(
        body,
        grid=(B, N // n_block),
        in_specs=[pl.BlockSpec((1, 1, T, F), lambda b, n: (b, 0, 0, 0))],
        out_specs=pl.BlockSpec((1, n_block, T, F), lambda b, n: (b, n, 0, 0)),
        out_shape=jax.ShapeDtypeStruct((B, N, T, F), jnp.float32),
    )(pos4)


def _tc_broadcast(pos, B, N, T, F):
    def body(pos_ref, out_ref):
        for t in range(T):
            row = pos_ref[0, t]
            out_ref[0, t] = jnp.broadcast_to(row[:, None], (F, N))

    out_t = pl.pallas_call(
        body,
        grid=(B,),
        in_specs=[pl.BlockSpec((1, T, F), lambda b: (b, 0, 0))],
        out_specs=pl.BlockSpec((1, T, F, N), lambda b: (b, 0, 0, 0)),
        out_shape=jax.ShapeDtypeStruct((B, T, F, N), jnp.float32),
    )(pos)
    return jnp.transpose(out_t, (0, 3, 1, 2))


def kernel(x, hours, pe):
    B, N, T, F = x.shape
    hours = hours.astype(jnp.int32)
    pos = _sc_gather(hours, pe, B, T, F)
    return _tc_broadcast(pos, B, N, T, F)


# speedup vs baseline: 5.3353x; 1.1093x over previous
import functools

import jax
import jax.numpy as jnp
from jax import lax
from jax.experimental import pallas as pl
from jax.experimental.pallas import tpu as pltpu
from jax.experimental.pallas import tpu_sc as plsc

_NC = 2
_NS = 16


def _sc_gather(hours_flat, pe, B, T, F):
    V = pe.shape[0]
    mesh = plsc.VectorSubcoreMesh(
        core_axis_name="c", subcore_axis_name="s", num_cores=_NC, num_subcores=_NS
    )

    @functools.partial(
        pl.kernel,
        out_type=jax.ShapeDtypeStruct((B * T, F), jnp.float32),
        mesh=mesh,
        scratch_types=[
            pltpu.VMEM((T,), jnp.int32),
            pltpu.VMEM((T, F), jnp.float32),
            pltpu.SemaphoreType.DMA,
        ],
        compiler_params=pltpu.CompilerParams(use_tc_tiling_on_sc=False),
    )
    def gather_kernel(hours_hbm, pe_hbm, pos_hbm, idx_v, rows_v, sem):
        wid = lax.axis_index("s") * _NC + lax.axis_index("c")

        @pl.when(wid < B)
        def _():
            pltpu.sync_copy(hours_hbm.at[pl.ds(wid * T, T)], idx_v)
            pltpu.async_copy(pe_hbm.at[idx_v], rows_v, sem).wait()
            pltpu.sync_copy(rows_v, pos_hbm.at[pl.ds(wid * T, T)])

    return gather_kernel(hours_flat, pe)


def _tc_broadcast(pos_flat, B, N, T, F):
    def body(pos_ref, out_ref):
        b = pl.program_id(0)
        for t2 in range(T // 2):
            pair = pos_ref[pl.ds(b * T * F + t2 * 2 * F, 2 * F)]
            out_ref[0, 2 * t2] = jnp.broadcast_to(pair[:F, None], (F, N))
            out_ref[0, 2 * t2 + 1] = jnp.broadcast_to(pair[F:, None], (F, N))

    out_t = pl.pallas_call(
        body,
        grid=(B,),
        in_specs=[pl.BlockSpec((B * T * F,), lambda b: (0,))],
        out_specs=pl.BlockSpec((1, T, F, N), lambda b: (b, 0, 0, 0)),
        out_shape=jax.ShapeDtypeStruct((B, T, F, N), jnp.float32),
    )(pos_flat)
    return jnp.transpose(out_t, (0, 3, 1, 2))


def kernel(x, hours, pe):
    B, N, T, F = x.shape
    hours_flat = hours.astype(jnp.int32).reshape(B * T)
    pos = _sc_gather(hours_flat, pe, B, T, F)
    return _tc_broadcast(pos.reshape(B * T * F), B, N, T, F)
